# baseline (speedup 1.0000x reference)
import functools

import jax
import jax.numpy as jnp
from jax import lax
from jax.experimental import pallas as pl
from jax.experimental.pallas import tpu as pltpu
from jax.experimental.pallas import tpu_sc as plsc

_N = 100000
_D = 128
_S = 256
_C = 80
_NCHUNK = _N // _C
_NCORE = 2
_NSUB = 16
_NW = _NCORE * _NSUB


def _sc_pool(x_hbm, idx_hbm, out_hbm, cnt_hbm,
             xbuf, ixbuf, onesbuf, zbuf, acc_sh, cnt_sh):
    ci = lax.axis_index("c")
    si = lax.axis_index("s")
    wid = si * _NCORE + ci

    ones16 = jnp.ones((16,), jnp.float32)
    zeros16 = jnp.zeros((16,), jnp.float32)

    def _fill_ones(r, carry):
        for j in range(_D // 16):
            onesbuf[r, pl.ds(16 * j, 16)] = ones16
        return carry
    lax.fori_loop(0, _C, _fill_ones, 0)

    def _fill_zero(r, carry):
        for j in range(_D // 16):
            zbuf[r, pl.ds(16 * j, 16)] = zeros16
        return carry
    lax.fori_loop(0, 64, _fill_zero, 0)

    @pl.when(si == 0)
    def _zero_shared():
        for k in range(_S // 64):
            pltpu.sync_copy(zbuf, acc_sh.at[pl.ds(64 * k, 64), :])
            pltpu.sync_copy(zbuf, cnt_sh.at[pl.ds(64 * k, 64), :])

    plsc.subcore_barrier()

    nk = (_NCHUNK - wid + _NW - 1) // _NW

    def _chunk(k, carry):
        base = (wid + k * _NW) * _C
        pltpu.sync_copy(x_hbm.at[pl.ds(base, _C), :], xbuf)
        pltpu.sync_copy(idx_hbm.at[pl.ds(base, _C)], ixbuf)
        pltpu.sync_copy(xbuf, acc_sh.at[ixbuf], add=True)
        pltpu.sync_copy(onesbuf, cnt_sh.at[ixbuf], add=True)
        return carry
    lax.fori_loop(0, nk, _chunk, 0)

    plsc.subcore_barrier()

    @pl.when(si == 0)
    def _writeout():
        pltpu.sync_copy(acc_sh, out_hbm.at[ci])
        pltpu.sync_copy(cnt_sh, cnt_hbm.at[ci])


_sc_call = pl.kernel(
    _sc_pool,
    out_type=(
        jax.ShapeDtypeStruct((_NCORE, _S, _D), jnp.float32),
        jax.ShapeDtypeStruct((_NCORE, _S, _D), jnp.float32),
    ),
    mesh=plsc.VectorSubcoreMesh(core_axis_name="c", subcore_axis_name="s"),
    scratch_types=[
        pltpu.VMEM((_C, _D), jnp.float32),
        pltpu.VMEM((_C,), jnp.int32),
        pltpu.VMEM((_C, _D), jnp.float32),
        pltpu.VMEM((64, _D), jnp.float32),
        pltpu.VMEM_SHARED((_S, _D), jnp.float32),
        pltpu.VMEM_SHARED((_S, _D), jnp.float32),
    ],
)


def _combine(part_ref, cnt_ref, out_ref):
    s = part_ref[0] + part_ref[1]
    c = cnt_ref[0, :, 0:1] + cnt_ref[1, :, 0:1]
    out_ref[...] = s / jnp.maximum(c, 1.0)


def kernel(x, batch_indices):
    idx32 = batch_indices.astype(jnp.int32)
    part, cnt = _sc_call(x, idx32)
    out = pl.pallas_call(
        _combine,
        out_shape=jax.ShapeDtypeStruct((_S, _D), jnp.float32),
    )(part, cnt)
    return (out, None)


# speedup vs baseline: 1.6191x; 1.6191x over previous
import jax
import jax.numpy as jnp
from jax import lax
from jax.experimental import pallas as pl
from jax.experimental.pallas import tpu as pltpu
from jax.experimental.pallas import tpu_sc as plsc

_N = 100000
_D = 128
_S = 256
_C = 80
_NCHUNK = _N // _C
_NCORE = 2
_NSUB = 16
_NW = _NCORE * _NSUB
_NKMAX = (_NCHUNK + _NW - 1) // _NW


def _sc_pool(x_hbm, idx_hbm, out_hbm, cnt_hbm,
             xbuf0, xbuf1, ixbuf0, ixbuf1, onesbuf, zbuf, acc_sh, cnt_sh,
             xsem0, xsem1, isem0, isem1):
    ci = lax.axis_index("c")
    si = lax.axis_index("s")
    wid = si * _NCORE + ci

    xbufs = (xbuf0, xbuf1)
    ixbufs = (ixbuf0, ixbuf1)
    xsems = (xsem0, xsem1)
    isems = (isem0, isem1)

    ones16 = jnp.ones((16,), jnp.float32)
    zeros16 = jnp.zeros((16,), jnp.float32)

    def _fill_ones(r, carry):
        for j in range(_D // 16):
            onesbuf[r, pl.ds(16 * j, 16)] = ones16
        return carry
    lax.fori_loop(0, _C, _fill_ones, 0)

    def _fill_zero(r, carry):
        for j in range(_D // 16):
            zbuf[r, pl.ds(16 * j, 16)] = zeros16
        return carry
    lax.fori_loop(0, 64, _fill_zero, 0)

    @pl.when(si == 0)
    def _zero_shared():
        for k in range(_S // 64):
            pltpu.sync_copy(zbuf, acc_sh.at[pl.ds(64 * k, 64), :])
            pltpu.sync_copy(zbuf, cnt_sh.at[pl.ds(64 * k, 64), :])

    plsc.subcore_barrier()

    nk = (_NCHUNK - wid + _NW - 1) // _NW

    def _start(j, b):
        base = (wid + j * _NW) * _C
        pltpu.make_async_copy(x_hbm.at[pl.ds(base, _C), :], xbufs[b],
                              xsems[b]).start()
        pltpu.make_async_copy(idx_hbm.at[pl.ds(base, _C)], ixbufs[b],
                              isems[b]).start()

    def _finish(b):
        pltpu.make_async_copy(x_hbm.at[pl.ds(0, _C), :], xbufs[b],
                              xsems[b]).wait()
        pltpu.make_async_copy(idx_hbm.at[pl.ds(0, _C)], ixbufs[b],
                              isems[b]).wait()
        pltpu.sync_copy(xbufs[b], acc_sh.at[ixbufs[b]], add=True)
        pltpu.sync_copy(onesbuf, cnt_sh.at[ixbufs[b]], add=True)

    _start(0, 0)

    def _pair(k2, carry):
        j0 = 2 * k2
        j1 = j0 + 1

        @pl.when(j0 < nk)
        def _even():
            @pl.when(j1 < nk)
            def _():
                _start(j1, 1)
            _finish(0)

        @pl.when(j1 < nk)
        def _odd():
            @pl.when(j1 + 1 < nk)
            def _():
                _start(j1 + 1, 0)
            _finish(1)

        return carry
    lax.fori_loop(0, (_NKMAX + 1) // 2, _pair, 0)

    plsc.subcore_barrier()

    @pl.when(si == 0)
    def _writeout():
        pltpu.sync_copy(acc_sh, out_hbm.at[ci])
        pltpu.sync_copy(cnt_sh, cnt_hbm.at[ci])


_sc_call = pl.kernel(
    _sc_pool,
    out_type=(
        jax.ShapeDtypeStruct((_NCORE, _S, _D), jnp.float32),
        jax.ShapeDtypeStruct((_NCORE, _S, _D), jnp.float32),
    ),
    mesh=plsc.VectorSubcoreMesh(core_axis_name="c", subcore_axis_name="s"),
    scratch_types=[
        pltpu.VMEM((_C, _D), jnp.float32),
        pltpu.VMEM((_C, _D), jnp.float32),
        pltpu.VMEM((_C,), jnp.int32),
        pltpu.VMEM((_C,), jnp.int32),
        pltpu.VMEM((_C, _D), jnp.float32),
        pltpu.VMEM((64, _D), jnp.float32),
        pltpu.VMEM_SHARED((_S, _D), jnp.float32),
        pltpu.VMEM_SHARED((_S, _D), jnp.float32),
        pltpu.SemaphoreType.DMA,
        pltpu.SemaphoreType.DMA,
        pltpu.SemaphoreType.DMA,
        pltpu.SemaphoreType.DMA,
    ],
)


def _combine(part_ref, cnt_ref, out_ref):
    s = part_ref[0] + part_ref[1]
    c = cnt_ref[0, :, 0:1] + cnt_ref[1, :, 0:1]
    out_ref[...] = s / jnp.maximum(c, 1.0)


def kernel(x, batch_indices):
    idx32 = batch_indices.astype(jnp.int32)
    part, cnt = _sc_call(x, idx32)
    out = pl.pallas_call(
        _combine,
        out_shape=jax.ShapeDtypeStruct((_S, _D), jnp.float32),
    )(part, cnt)
    return (out, None)


# speedup vs baseline: 2.0759x; 1.2821x over previous
import jax
import jax.numpy as jnp
from jax import lax
from jax.experimental import pallas as pl
from jax.experimental.pallas import tpu as pltpu
from jax.experimental.pallas import tpu_sc as plsc

_N = 100000
_D = 128
_S = 256
_C = 80
_NCHUNK = _N // _C
_NCORE = 2
_NSUB = 16
_NW = _NCORE * _NSUB
_NKMAX = (_NCHUNK + _NW - 1) // _NW
_PAD = 16


def _sc_pool(x_hbm, idx_hbm, idxp_hbm, out_hbm, lb_hbm, ub_hbm,
             xbuf0, xbuf1, ixbuf0, ixbuf1, ixp0, ixp1, zbuf,
             lb_loc, ub_loc, acc_sh,
             xsem0, xsem1, isem0, isem1, psem0, psem1):
    ci = lax.axis_index("c")
    si = lax.axis_index("s")
    wid = si * _NCORE + ci

    xbufs = (xbuf0, xbuf1)
    ixbufs = (ixbuf0, ixbuf1)
    ixps = (ixp0, ixp1)
    xsems = (xsem0, xsem1)
    isems = (isem0, isem1)
    psems = (psem0, psem1)

    zeros16 = jnp.zeros((16,), jnp.float32)
    zeros16i = jnp.zeros((16,), jnp.int32)
    lane = lax.iota(jnp.int32, 16)

    def _fill_zero(r, carry):
        for j in range(_D // 16):
            zbuf[r, pl.ds(16 * j, 16)] = zeros16
        return carry
    lax.fori_loop(0, 64, _fill_zero, 0)

    for g in range(_S // 16):
        lb_loc[pl.ds(16 * g, 16)] = zeros16i
        ub_loc[pl.ds(16 * g, 16)] = zeros16i

    @pl.when(si == 0)
    def _zero_shared():
        for k in range(_S // 64):
            pltpu.sync_copy(zbuf, acc_sh.at[pl.ds(64 * k, 64), :])

    plsc.subcore_barrier()

    nk = (_NCHUNK - wid + _NW - 1) // _NW

    def _start(j, b):
        base = (wid + j * _NW) * _C
        pltpu.make_async_copy(x_hbm.at[pl.ds(base, _C), :], xbufs[b],
                              xsems[b]).start()
        pltpu.make_async_copy(idx_hbm.at[pl.ds(base, _C)], ixbufs[b],
                              isems[b]).start()
        pltpu.make_async_copy(idxp_hbm.at[pl.ds(base, _C + _PAD)], ixps[b],
                              psems[b]).start()

    def _finish(j, b):
        base = (wid + j * _NW) * _C
        pltpu.make_async_copy(x_hbm.at[pl.ds(0, _C), :], xbufs[b],
                              xsems[b]).wait()
        pltpu.make_async_copy(idx_hbm.at[pl.ds(0, _C)], ixbufs[b],
                              isems[b]).wait()
        pltpu.make_async_copy(idxp_hbm.at[pl.ds(0, _C + _PAD)], ixps[b],
                              psems[b]).wait()
        pltpu.sync_copy(xbufs[b], acc_sh.at[ixbufs[b]], add=True)
        for r in range(_C // 16):
            cur = ixps[b][pl.ds(_PAD + 16 * r, 16)]
            prv = ixps[b][pl.ds(_PAD - 1 + 16 * r, 16)]
            trans = cur != prv
            pos = base + 16 * r + lane
            plsc.store_scatter(lb_loc, [cur], pos, mask=trans)
            plsc.store_scatter(ub_loc, [prv], pos, mask=trans)

        @pl.when(base + _C == _N)
        def _last_chunk():
            cur = ixps[b][pl.ds(_PAD + _C - 16, 16)]
            endv = jnp.full((16,), _N, jnp.int32)
            plsc.store_scatter(ub_loc, [cur], endv, mask=lane == 15)

    _start(0, 0)

    def _pair(k2, carry):
        j0 = 2 * k2
        j1 = j0 + 1

        @pl.when(j0 < nk)
        def _even():
            @pl.when(j1 < nk)
            def _():
                _start(j1, 1)
            _finish(j0, 0)

        @pl.when(j1 < nk)
        def _odd():
            @pl.when(j1 + 1 < nk)
            def _():
                _start(j1 + 1, 0)
            _finish(j1, 1)

        return carry
    lax.fori_loop(0, (_NKMAX + 1) // 2, _pair, 0)

    pltpu.sync_copy(lb_loc, lb_hbm.at[wid])
    pltpu.sync_copy(ub_loc, ub_hbm.at[wid])

    plsc.subcore_barrier()

    @pl.when(si == 0)
    def _writeout():
        pltpu.sync_copy(acc_sh, out_hbm.at[ci])


_sc_call = pl.kernel(
    _sc_pool,
    out_type=(
        jax.ShapeDtypeStruct((_NCORE, _S, _D), jnp.float32),
        jax.ShapeDtypeStruct((_NW, _S), jnp.int32),
        jax.ShapeDtypeStruct((_NW, _S), jnp.int32),
    ),
    mesh=plsc.VectorSubcoreMesh(core_axis_name="c", subcore_axis_name="s"),
    compiler_params=pltpu.CompilerParams(needs_layout_passes=False),
    scratch_types=[
        pltpu.VMEM((_C, _D), jnp.float32),
        pltpu.VMEM((_C, _D), jnp.float32),
        pltpu.VMEM((_C,), jnp.int32),
        pltpu.VMEM((_C,), jnp.int32),
        pltpu.VMEM((_C + _PAD,), jnp.int32),
        pltpu.VMEM((_C + _PAD,), jnp.int32),
        pltpu.VMEM((64, _D), jnp.float32),
        pltpu.VMEM((_S,), jnp.int32),
        pltpu.VMEM((_S,), jnp.int32),
        pltpu.VMEM_SHARED((_S, _D), jnp.float32),
        pltpu.SemaphoreType.DMA,
        pltpu.SemaphoreType.DMA,
        pltpu.SemaphoreType.DMA,
        pltpu.SemaphoreType.DMA,
        pltpu.SemaphoreType.DMA,
        pltpu.SemaphoreType.DMA,
    ],
)


def _combine(part_ref, lb_ref, ub_ref, out_ref):
    s = part_ref[0] + part_ref[1]
    diff = (ub_ref[...] - lb_ref[...]).astype(jnp.float32)
    ones = jnp.ones((_NW, 1), jnp.float32)
    c = lax.dot_general(diff, ones, (((0,), (0,)), ((), ())),
                        precision=lax.Precision.HIGHEST,
                        preferred_element_type=jnp.float32)
    out_ref[...] = s / jnp.maximum(c, 1.0)


def kernel(x, batch_indices):
    idx32 = batch_indices.astype(jnp.int32)
    idxp = jnp.concatenate([jnp.zeros((_PAD,), jnp.int32), idx32])
    part, lb, ub = _sc_call(x, idx32, idxp)
    out = pl.pallas_call(
        _combine,
        out_shape=jax.ShapeDtypeStruct((_S, _D), jnp.float32),
    )(part, lb, ub)
    return (out, None)


# speedup vs baseline: 2.3522x; 1.1331x over previous
import jax
import jax.numpy as jnp
from jax import lax
from jax.experimental import pallas as pl
from jax.experimental.pallas import tpu as pltpu
from jax.experimental.pallas import tpu_sc as plsc

_N = 100000
_D = 128
_S = 256
_NSC = 40000
_C = 80
_NCHUNK = _NSC // _C
_CB = 400
_NBCHUNK = (_N - _NSC) // _CB
_NCORE = 2
_NSUB = 16
_NW = _NCORE * _NSUB
_NKMAX = (_NCHUNK + _NW - 1) // _NW
_NBMAX = (_NBCHUNK + _NW - 1) // _NW
_PAD = 16

_BLK = 2000
_NTCB = (_N - _NSC) // _BLK


def _sc_pool(x_hbm, idx_hbm, idxp_hbm, out_hbm, lb_hbm, ub_hbm,
             xbuf0, xbuf1, ixbuf0, ixbuf1, ixp0, ixp1, bbuf, zbuf,
             lb_loc, ub_loc, acc_sh,
             xsem0, xsem1, isem0, isem1, psem0, psem1, ssem0, ssem1):
    ci = lax.axis_index("c")
    si = lax.axis_index("s")
    wid = si * _NCORE + ci

    xbufs = (xbuf0, xbuf1)
    ixbufs = (ixbuf0, ixbuf1)
    ixps = (ixp0, ixp1)
    xsems = (xsem0, xsem1)
    isems = (isem0, isem1)
    psems = (psem0, psem1)
    ssems = (ssem0, ssem1)

    zeros16 = jnp.zeros((16,), jnp.float32)
    zeros16i = jnp.zeros((16,), jnp.int32)
    lane = lax.iota(jnp.int32, 16)

    def _fill_zero(r, carry):
        for j in range(_D // 16):
            zbuf[r, pl.ds(16 * j, 16)] = zeros16
        return carry
    lax.fori_loop(0, 64, _fill_zero, 0)

    for g in range(_S // 16):
        lb_loc[pl.ds(16 * g, 16)] = zeros16i
        ub_loc[pl.ds(16 * g, 16)] = zeros16i

    @pl.when(si == 0)
    def _zero_shared():
        for k in range(_S // 64):
            pltpu.sync_copy(zbuf, acc_sh.at[pl.ds(64 * k, 64), :])

    plsc.subcore_barrier()

    nk = (_NCHUNK - wid + _NW - 1) // _NW

    def _start(j, b):
        base = (wid + j * _NW) * _C

        @pl.when(j >= 2)
        def _wait_prev_scatter():
            pltpu.make_async_copy(xbufs[b], acc_sh.at[ixbufs[b]],
                                  ssems[b]).wait()

        pltpu.make_async_copy(x_hbm.at[pl.ds(base, _C), :], xbufs[b],
                              xsems[b]).start()
        pltpu.make_async_copy(idx_hbm.at[pl.ds(base, _C)], ixbufs[b],
                              isems[b]).start()
        pltpu.make_async_copy(idxp_hbm.at[pl.ds(base, _C + _PAD)], ixps[b],
                              psems[b]).start()

    def _finish(j, b):
        base = (wid + j * _NW) * _C
        pltpu.make_async_copy(x_hbm.at[pl.ds(0, _C), :], xbufs[b],
                              xsems[b]).wait()
        pltpu.make_async_copy(idx_hbm.at[pl.ds(0, _C)], ixbufs[b],
                              isems[b]).wait()
        pltpu.make_async_copy(idxp_hbm.at[pl.ds(0, _C + _PAD)], ixps[b],
                              psems[b]).wait()
        pltpu.make_async_copy(xbufs[b], acc_sh.at[ixbufs[b]],
                              ssems[b]).start(add=True)
        for r in range(_C // 16):
            cur = ixps[b][pl.ds(_PAD + 16 * r, 16)]
            prv = ixps[b][pl.ds(_PAD - 1 + 16 * r, 16)]
            trans = cur != prv
            pos = base + 16 * r + lane
            plsc.store_scatter(lb_loc, [cur], pos, mask=trans)
            plsc.store_scatter(ub_loc, [prv], pos, mask=trans)

    _start(0, 0)

    def _pair(k2, carry):
        j0 = 2 * k2
        j1 = j0 + 1

        @pl.when(j0 < nk)
        def _even():
            @pl.when(j1 < nk)
            def _():
                _start(j1, 1)
            _finish(j0, 0)

        @pl.when(j1 < nk)
        def _odd():
            @pl.when(j1 + 1 < nk)
            def _():
                _start(j1 + 1, 0)
            _finish(j1, 1)

        return carry
    lax.fori_loop(0, (_NKMAX + 1) // 2, _pair, 0)

    for b in range(2):
        pltpu.make_async_copy(xbufs[b], acc_sh.at[ixbufs[b]],
                              ssems[b]).wait()

    nb = (_NBCHUNK - wid + _NW - 1) // _NW

    def _bchunk(k, carry):
        base = _NSC + (wid + k * _NW) * _CB
        pltpu.sync_copy(idxp_hbm.at[pl.ds(base, _CB + _PAD)], bbuf)

        def _bvec(r, carry2):
            cur = bbuf[pl.ds(_PAD + 16 * r, 16)]
            prv = bbuf[pl.ds(_PAD - 1 + 16 * r, 16)]
            trans = cur != prv
            pos = base + 16 * r + lane
            plsc.store_scatter(lb_loc, [cur], pos, mask=trans)
            plsc.store_scatter(ub_loc, [prv], pos, mask=trans)
            return carry2
        lax.fori_loop(0, _CB // 16, _bvec, 0)
        return carry
    lax.fori_loop(0, nb, _bchunk, 0)

    @pl.when(wid == (_NBCHUNK - 1) % _NW)
    def _last_chunk():
        cur = bbuf[pl.ds(_PAD + _CB - 16, 16)]
        endv = jnp.full((16,), _N, jnp.int32)
        plsc.store_scatter(ub_loc, [cur], endv, mask=lane == 15)

    pltpu.sync_copy(lb_loc, lb_hbm.at[wid])
    pltpu.sync_copy(ub_loc, ub_hbm.at[wid])

    plsc.subcore_barrier()

    @pl.when(si == 0)
    def _writeout():
        pltpu.sync_copy(acc_sh, out_hbm.at[ci])


_sc_call = pl.kernel(
    _sc_pool,
    out_type=(
        jax.ShapeDtypeStruct((_NCORE, _S, _D), jnp.float32),
        jax.ShapeDtypeStruct((_NW, _S), jnp.int32),
        jax.ShapeDtypeStruct((_NW, _S), jnp.int32),
    ),
    mesh=plsc.VectorSubcoreMesh(core_axis_name="c", subcore_axis_name="s"),
    compiler_params=pltpu.CompilerParams(needs_layout_passes=False),
    scratch_types=[
        pltpu.VMEM((_C, _D), jnp.float32),
        pltpu.VMEM((_C, _D), jnp.float32),
        pltpu.VMEM((_C,), jnp.int32),
        pltpu.VMEM((_C,), jnp.int32),
        pltpu.VMEM((_C + _PAD,), jnp.int32),
        pltpu.VMEM((_C + _PAD,), jnp.int32),
        pltpu.VMEM((_CB + _PAD,), jnp.int32),
        pltpu.VMEM((64, _D), jnp.float32),
        pltpu.VMEM((_S,), jnp.int32),
        pltpu.VMEM((_S,), jnp.int32),
        pltpu.VMEM_SHARED((_S, _D), jnp.float32),
        pltpu.SemaphoreType.DMA,
        pltpu.SemaphoreType.DMA,
        pltpu.SemaphoreType.DMA,
        pltpu.SemaphoreType.DMA,
        pltpu.SemaphoreType.DMA,
        pltpu.SemaphoreType.DMA,
        pltpu.SemaphoreType.DMA,
        pltpu.SemaphoreType.DMA,
    ],
)


def _tc_pool(idx_ref, x_ref, out_ref, acc_ref):
    i = pl.program_id(0)

    idx = idx_ref[0, 0, :]
    xb = x_ref[...].astype(jnp.bfloat16)
    seg = lax.broadcasted_iota(jnp.int32, (_S, _BLK), 0)
    oh = (seg == idx[None, :]).astype(jnp.bfloat16)
    psum = lax.dot(oh, xb, preferred_element_type=jnp.float32)

    @pl.when(i == 0)
    def _init():
        acc_ref[...] = psum

    @pl.when(i > 0)
    def _acc():
        acc_ref[...] += psum

    @pl.when(i == _NTCB - 1)
    def _fin():
        out_ref[...] = acc_ref[...]


def _combine(tc_ref, sc_ref, lb_ref, ub_ref, out_ref):
    s = tc_ref[...] + sc_ref[0] + sc_ref[1]
    diff = (ub_ref[...] - lb_ref[...]).astype(jnp.float32)
    ones = jnp.ones((_NW, 1), jnp.float32)
    c = lax.dot_general(diff, ones, (((0,), (0,)), ((), ())),
                        precision=lax.Precision.HIGHEST,
                        preferred_element_type=jnp.float32)
    out_ref[...] = s / jnp.maximum(c, 1.0)


def kernel(x, batch_indices):
    idx32 = batch_indices.astype(jnp.int32)
    idxp = jnp.concatenate([jnp.zeros((_PAD,), jnp.int32), idx32])
    sc_part, lb, ub = _sc_call(x, idx32, idxp)

    idx3 = idx32[_NSC:].reshape(_NTCB, 1, _BLK)
    tc_part = pl.pallas_call(
        _tc_pool,
        grid=(_NTCB,),
        in_specs=[
            pl.BlockSpec((1, 1, _BLK), lambda i: (i, 0, 0)),
            pl.BlockSpec((_BLK, _D), lambda i: (i + _NSC // _BLK, 0)),
        ],
        out_specs=pl.BlockSpec((_S, _D), lambda i: (0, 0)),
        out_shape=jax.ShapeDtypeStruct((_S, _D), jnp.float32),
        scratch_shapes=[pltpu.VMEM((_S, _D), jnp.float32)],
    )(idx3, x)

    out = pl.pallas_call(
        _combine,
        out_shape=jax.ShapeDtypeStruct((_S, _D), jnp.float32),
    )(tc_part, sc_part, lb, ub)
    return (out, None)
